# baseline (speedup 1.0000x reference)
import functools

import jax
import jax.numpy as jnp
from jax import lax
from jax.experimental import pallas as pl
from jax.experimental.pallas import tpu as pltpu
from jax.experimental.pallas import tpu_sc as plsc

_LANES = 16
_IDXW = 128
_PACK = 4
_SUB = 128


def _sc_scores(user_embed4, book_embed4, uids, pids, nids, B, K, D):
    info = plsc.get_sparse_core_info()
    NC, NS = info.num_cores, info.num_subcores
    NW = NC * NS
    chunk = B // NW
    n_sub = chunk // _SUB
    n_grp = _SUB // _LANES
    n_iu = chunk // _IDXW
    n_in = chunk * K // _IDXW
    W = _PACK * D

    mesh = plsc.VectorSubcoreMesh(core_axis_name="c", subcore_axis_name="s")

    @functools.partial(
        pl.kernel,
        out_type=[
            jax.ShapeDtypeStruct((B,), jnp.float32),
            jax.ShapeDtypeStruct((B * K,), jnp.float32),
        ],
        mesh=mesh,
        compiler_params=pltpu.CompilerParams(
            needs_layout_passes=False,
        ),
        scratch_types=[
            pltpu.VMEM((n_iu, _IDXW), jnp.int32),
            pltpu.VMEM((n_iu, _IDXW), jnp.int32),
            pltpu.VMEM((n_in, _IDXW), jnp.int32),
            pltpu.VMEM((n_iu, _IDXW), jnp.int32),
            pltpu.VMEM((n_iu, _IDXW), jnp.int32),
            pltpu.VMEM((n_in, _IDXW), jnp.int32),
            pltpu.VMEM((chunk,), jnp.int32),
            pltpu.VMEM((chunk,), jnp.int32),
            pltpu.VMEM((chunk * K,), jnp.int32),
            pltpu.VMEM((_SUB, W), jnp.float32),
            pltpu.VMEM((_SUB, W), jnp.float32),
            pltpu.VMEM((_SUB * K, W), jnp.float32),
            pltpu.VMEM((chunk,), jnp.float32),
            pltpu.VMEM((chunk * K,), jnp.float32),
            pltpu.SemaphoreType.DMA,
        ],
    )
    def sc_kernel(uids_h, pids_h, nids_h, uemb_h, bemb_h, pos_o, neg_o,
                  idx_u, idx_p, idx_n, q_u, q_p, q_n, o_u, o_p, o_n,
                  rows_u, rows_p, rows_n, pos_v, neg_v, sem):
        wid = lax.axis_index("s") * NC + lax.axis_index("c")
        pltpu.sync_copy(uids_h.at[wid], idx_u)
        pltpu.sync_copy(pids_h.at[wid], idx_p)
        pltpu.sync_copy(nids_h.at[wid], idx_n)

        def split_ids(idx_ref, q_ref, o_ref, n_rows):
            def body(i, carry):
                r = i // (_IDXW // _LANES)
                c = (i % (_IDXW // _LANES)) * _LANES
                v = idx_ref[r, pl.ds(c, _LANES)]
                q_ref[r, pl.ds(c, _LANES)] = lax.shift_right_logical(v, 2)
                flat = i * _LANES
                o_ref[pl.ds(flat, _LANES)] = lax.bitwise_and(v, _PACK - 1) * D
                return carry
            lax.fori_loop(0, n_rows * (_IDXW // _LANES), body, 0)

        split_ids(idx_u, q_u, o_u, n_iu)
        split_ids(idx_p, q_p, o_p, n_iu)
        split_ids(idx_n, q_n, o_n, n_in)

        for s in range(n_sub):
            cps = [
                pltpu.async_copy(uemb_h.at[q_u.at[s]], rows_u, sem),
                pltpu.async_copy(bemb_h.at[q_p.at[s]], rows_p, sem),
            ]
            for j in range(K):
                r = s * K + j
                cps.append(pltpu.async_copy(
                    bemb_h.at[q_n.at[r]],
                    rows_n.at[pl.ds(j * _IDXW, _IDXW)], sem))
            for c in cps:
                c.wait()

            def group(g, carry):
                bloc = g * _LANES + lax.iota(jnp.int32, _LANES)
                babs = s * _SUB + bloc
                ou = o_u[pl.ds(s * _SUB + g * _LANES, _LANES)]
                op = o_p[pl.ds(s * _SUB + g * _LANES, _LANES)]
                accp = jnp.zeros((_LANES,), jnp.float32)
                accn = [jnp.zeros((_LANES,), jnp.float32) for _ in range(K)]
                onk = [plsc.load_gather(
                    o_n, [(babs * K + k)]) for k in range(K)]
                nrow = [bloc * K + k for k in range(K)]
                for d in range(D):
                    uv = plsc.load_gather(rows_u, [bloc, ou + d])
                    pv = plsc.load_gather(rows_p, [bloc, op + d])
                    accp = accp + uv * pv
                    for k in range(K):
                        nv = plsc.load_gather(rows_n, [nrow[k], onk[k] + d])
                        accn[k] = accn[k] + uv * nv
                plsc.store_scatter(pos_v, [babs], accp)
                for k in range(K):
                    plsc.store_scatter(neg_v, [babs * K + k], accn[k])
                return carry

            lax.fori_loop(0, n_grp, group, 0)

        pltpu.sync_copy(pos_v, pos_o.at[pl.ds(wid * chunk, chunk)])
        pltpu.sync_copy(neg_v, neg_o.at[pl.ds(wid * chunk * K, chunk * K)])

    return sc_kernel(uids, pids, nids, user_embed4, book_embed4)


def _loss_tc(pos_s, neg_s, B):
    pos2 = pos_s.reshape(-1, 128)
    neg2 = neg_s.reshape(-1, 128)

    def body(p_ref, n_ref, o_ref):
        p = p_ref[...]
        n = n_ref[...]
        lp = jnp.log(1.0 / (1.0 + jnp.exp(-p)) + 1e-10)
        ln = jnp.log(1.0 / (1.0 + jnp.exp(n)) + 1e-10)
        o_ref[0, 0] = -(jnp.sum(lp) + jnp.sum(ln)) * (1.0 / B)

    out = pl.pallas_call(
        body,
        out_shape=jax.ShapeDtypeStruct((1, 1), jnp.float32),
        out_specs=pl.BlockSpec(memory_space=pltpu.SMEM),
    )(pos2, neg2)
    return out[0, 0]


def kernel(user_embed, book_embed, user_ids, pos_book_ids, neg_book_ids):
    B = user_ids.shape[0]
    K = neg_book_ids.shape[1]
    D = user_embed.shape[1]
    info = plsc.get_sparse_core_info()
    NW = info.num_cores * info.num_subcores
    uids = user_ids.astype(jnp.int32).reshape(NW, -1, _IDXW)
    pids = pos_book_ids.astype(jnp.int32).reshape(NW, -1, _IDXW)
    nids = neg_book_ids.astype(jnp.int32).reshape(NW, -1, _IDXW)
    user4 = user_embed.reshape(-1, _PACK * D)
    book4 = book_embed.reshape(-1, _PACK * D)
    pos_s, neg_s = _sc_scores(user4, book4, uids, pids, nids, B, K, D)
    return _loss_tc(pos_s, neg_s, B)


# speedup vs baseline: 1.0187x; 1.0187x over previous
import functools

import jax
import jax.numpy as jnp
from jax import lax
from jax.experimental import pallas as pl
from jax.experimental.pallas import tpu as pltpu
from jax.experimental.pallas import tpu_sc as plsc

_LANES = 16
_IDXW = 128
_W = 128
_SUB = 128


def _sc_scores(user_pad, book_pad, uids, pids, nids, B, K, D):
    info = plsc.get_sparse_core_info()
    NC, NS = info.num_cores, info.num_subcores
    NW = NC * NS
    chunk = B // NW
    n_sub = chunk // _SUB
    n_grp = _SUB // _LANES
    n_iu = chunk // _IDXW
    n_in = chunk * K // _IDXW

    mesh = plsc.VectorSubcoreMesh(core_axis_name="c", subcore_axis_name="s")

    @functools.partial(
        pl.kernel,
        out_type=[
            jax.ShapeDtypeStruct((B,), jnp.float32),
            jax.ShapeDtypeStruct((B * K,), jnp.float32),
        ],
        mesh=mesh,
        compiler_params=pltpu.CompilerParams(
            needs_layout_passes=False,
        ),
        scratch_types=[
            pltpu.VMEM((n_iu, _IDXW), jnp.int32),
            pltpu.VMEM((n_iu, _IDXW), jnp.int32),
            pltpu.VMEM((n_in, _IDXW), jnp.int32),
            pltpu.VMEM((_SUB, _W), jnp.float32),
            pltpu.VMEM((_SUB, _W), jnp.float32),
            pltpu.VMEM((_SUB * K, _W), jnp.float32),
            pltpu.VMEM((chunk,), jnp.float32),
            pltpu.VMEM((chunk * K,), jnp.float32),
            pltpu.SemaphoreType.DMA,
        ],
    )
    def sc_kernel(uids_h, pids_h, nids_h, uemb_h, bemb_h, pos_o, neg_o,
                  idx_u, idx_p, idx_n, rows_u, rows_p, rows_n, pos_v, neg_v,
                  sem):
        wid = lax.axis_index("s") * NC + lax.axis_index("c")
        pltpu.sync_copy(uids_h.at[wid], idx_u)
        pltpu.sync_copy(pids_h.at[wid], idx_p)
        pltpu.sync_copy(nids_h.at[wid], idx_n)

        for s in range(n_sub):
            cps = [
                pltpu.async_copy(uemb_h.at[idx_u.at[s]], rows_u, sem),
                pltpu.async_copy(bemb_h.at[idx_p.at[s]], rows_p, sem),
            ]
            for j in range(K):
                cps.append(pltpu.async_copy(
                    bemb_h.at[idx_n.at[s * K + j]],
                    rows_n.at[pl.ds(j * _IDXW, _IDXW)], sem))
            for c in cps:
                c.wait()

            def group(g, carry):
                bloc = g * _LANES + lax.iota(jnp.int32, _LANES)
                babs = s * _SUB + bloc
                accp = jnp.zeros((_LANES,), jnp.float32)
                accn = [jnp.zeros((_LANES,), jnp.float32) for _ in range(K)]
                nrow = [bloc * K + k for k in range(K)]
                for d in range(D):
                    col = jnp.full((_LANES,), d, jnp.int32)
                    uv = plsc.load_gather(rows_u, [bloc, col])
                    pv = plsc.load_gather(rows_p, [bloc, col])
                    accp = accp + uv * pv
                    for k in range(K):
                        nv = plsc.load_gather(rows_n, [nrow[k], col])
                        accn[k] = accn[k] + uv * nv
                plsc.store_scatter(pos_v, [babs], accp)
                for k in range(K):
                    plsc.store_scatter(neg_v, [babs * K + k], accn[k])
                return carry

            lax.fori_loop(0, n_grp, group, 0)

        pltpu.sync_copy(pos_v, pos_o.at[pl.ds(wid * chunk, chunk)])
        pltpu.sync_copy(neg_v, neg_o.at[pl.ds(wid * chunk * K, chunk * K)])

    return sc_kernel(uids, pids, nids, user_pad, book_pad)


def _loss_tc(pos_s, neg_s, B):
    pos2 = pos_s.reshape(-1, 128)
    neg2 = neg_s.reshape(-1, 128)

    def body(p_ref, n_ref, o_ref):
        p = p_ref[...]
        n = n_ref[...]
        lp = jnp.log(1.0 / (1.0 + jnp.exp(-p)) + 1e-10)
        ln = jnp.log(1.0 / (1.0 + jnp.exp(n)) + 1e-10)
        o_ref[0, 0] = -(jnp.sum(lp) + jnp.sum(ln)) * (1.0 / B)

    out = pl.pallas_call(
        body,
        out_shape=jax.ShapeDtypeStruct((1, 1), jnp.float32),
        out_specs=pl.BlockSpec(memory_space=pltpu.SMEM),
    )(pos2, neg2)
    return out[0, 0]


def kernel(user_embed, book_embed, user_ids, pos_book_ids, neg_book_ids):
    B = user_ids.shape[0]
    K = neg_book_ids.shape[1]
    D = user_embed.shape[1]
    info = plsc.get_sparse_core_info()
    NW = info.num_cores * info.num_subcores
    uids = user_ids.astype(jnp.int32).reshape(NW, -1, _IDXW)
    pids = pos_book_ids.astype(jnp.int32).reshape(NW, -1, _IDXW)
    nids = neg_book_ids.astype(jnp.int32).reshape(NW, -1, _IDXW)
    user_pad = jnp.pad(user_embed, ((0, 0), (0, _W - D)))
    book_pad = jnp.pad(book_embed, ((0, 0), (0, _W - D)))
    pos_s, neg_s = _sc_scores(user_pad, book_pad, uids, pids, nids, B, K, D)
    return _loss_tc(pos_s, neg_s, B)


# speedup vs baseline: 1.3555x; 1.3306x over previous
import functools

import jax
import jax.numpy as jnp
from jax import lax
from jax.experimental import pallas as pl
from jax.experimental.pallas import tpu as pltpu
from jax.experimental.pallas import tpu_sc as plsc

_LANES = 16
_IDXW = 128
_W = 128
_SUB = 128
_BLK = 1024


def _sc_gather_book(book_t, allids, n_ids, V, D):
    info = plsc.get_sparse_core_info()
    NC, NS = info.num_cores, info.num_subcores
    NW = NC * NS
    n_chunks = (V + _BLK - 1) // _BLK
    V_pad = ((V + 127) // 128) * 128
    n_piece = n_ids // 8192
    HMAX = 4112
    TMAX = 656

    mesh = plsc.VectorSubcoreMesh(core_axis_name="c", subcore_axis_name="s")

    @functools.partial(
        pl.kernel,
        out_type=jax.ShapeDtypeStruct((n_ids + NW, _W), jnp.float32),
        mesh=mesh,
        compiler_params=pltpu.CompilerParams(
            needs_layout_passes=False,
        ),
        scratch_types=[
            pltpu.VMEM((64, _IDXW), jnp.int32),
            pltpu.VMEM((D, _BLK), jnp.float32),
            pltpu.VMEM((HMAX,), jnp.int32),
            pltpu.VMEM((HMAX,), jnp.int32),
            pltpu.VMEM((TMAX,), jnp.int32),
            pltpu.VMEM((TMAX,), jnp.int32),
            pltpu.VMEM((1, _IDXW), jnp.int32),
            pltpu.VMEM((_IDXW, _W), jnp.float32),
            pltpu.SemaphoreType.DMA,
        ],
    )
    def phase_a(ids_h, bt_h, out_h, idbuf, blk, hv, hs, tv, ts, sidx, ebuf,
                sem):
        wid = lax.axis_index("s") * NC + lax.axis_index("c")
        iota16 = lax.iota(jnp.int32, _LANES)
        start_c = (wid * n_chunks) // NW
        end_c = ((wid + 1) * n_chunks) // NW
        lo_w = start_c * _BLK
        hi_w = jnp.minimum(end_c * _BLK, V)

        def scan_piece(p, cnt):
            pltpu.sync_copy(ids_h.at[pl.ds(p * 64, 64)], idbuf)

            def scan_vec(i, cnt):
                r = i // 8
                c = (i % 8) * _LANES
                v = idbuf[r, pl.ds(c, _LANES)]
                m = jnp.logical_and(v >= lo_w, v < hi_w)
                plsc.store_compressed(hv.at[pl.ds(cnt, _LANES)], v, mask=m)
                slot = p * 8192 + i * _LANES + iota16
                plsc.store_compressed(hs.at[pl.ds(cnt, _LANES)], slot, mask=m)
                return cnt + jnp.max(plsc.all_reduce_population_count(m))

            return lax.fori_loop(0, 512, scan_vec, cnt)

        cnt = lax.fori_loop(0, n_piece, scan_piece, jnp.int32(0))
        hv[pl.ds(cnt, _LANES)] = jnp.full((_LANES,), -1, jnp.int32)
        n_hvec = (cnt + _LANES - 1) // _LANES

        def do_chunk(c, carry):
            lo = (start_c + c) * _BLK
            hi = jnp.minimum(lo + _BLK, V)
            win = jnp.minimum(lo, jnp.int32(V_pad - _BLK))
            pltpu.sync_copy(bt_h.at[:, pl.ds(win, _BLK)], blk)

            def resc(i, c2):
                v = hv[pl.ds(i * _LANES, _LANES)]
                s = hs[pl.ds(i * _LANES, _LANES)]
                m = jnp.logical_and(v >= lo, v < hi)
                plsc.store_compressed(
                    tv.at[pl.ds(c2, _LANES)], v - win, mask=m)
                plsc.store_compressed(ts.at[pl.ds(c2, _LANES)], s, mask=m)
                return c2 + jnp.max(plsc.all_reduce_population_count(m))

            cnt2 = lax.fori_loop(0, n_hvec, resc, jnp.int32(0))

            def padv(j, carry):
                tv[pl.ds(cnt2 + j * _LANES, _LANES)] = jnp.zeros(
                    (_LANES,), jnp.int32)
                ts[pl.ds(cnt2 + j * _LANES, _LANES)] = (
                    jnp.zeros((_LANES,), jnp.int32) + (n_ids + wid))
                return carry

            lax.fori_loop(0, 8, padv, 0)

            def batch(b, carry):
                def sub(j, carry):
                    off = b * _IDXW + j * _LANES
                    vloc = tv[pl.ds(off, _LANES)]
                    sidx[0, pl.ds(j * _LANES, _LANES)] = ts[
                        pl.ds(off, _LANES)]
                    lane = j * _LANES + iota16
                    for d in range(D):
                        dv = jnp.full((_LANES,), d, jnp.int32)
                        val = plsc.load_gather(blk, [dv, vloc])
                        plsc.store_scatter(ebuf, [lane, dv], val)
                    return carry

                lax.fori_loop(0, _IDXW // _LANES, sub, 0)
                pltpu.async_copy(ebuf, out_h.at[sidx.at[0]], sem).wait()
                return carry

            lax.fori_loop(0, (cnt2 + _IDXW - 1) // _IDXW, batch, 0)
            return carry

        lax.fori_loop(0, end_c - start_c, do_chunk, 0)

    return phase_a(allids, book_t)


def _sc_scores(rows_book, user_pad, uids, B, K, D):
    info = plsc.get_sparse_core_info()
    NC, NS = info.num_cores, info.num_subcores
    NW = NC * NS
    chunk = B // NW
    n_sub = chunk // _SUB
    n_grp = _SUB // _LANES
    n_iu = chunk // _IDXW

    mesh = plsc.VectorSubcoreMesh(core_axis_name="c", subcore_axis_name="s")

    @functools.partial(
        pl.kernel,
        out_type=[
            jax.ShapeDtypeStruct((B,), jnp.float32),
            jax.ShapeDtypeStruct((B * K,), jnp.float32),
        ],
        mesh=mesh,
        compiler_params=pltpu.CompilerParams(
            needs_layout_passes=False,
        ),
        scratch_types=[
            pltpu.VMEM((n_iu, _IDXW), jnp.int32),
            pltpu.VMEM((_SUB, _W), jnp.float32),
            pltpu.VMEM((_SUB, _W), jnp.float32),
            pltpu.VMEM((_SUB * K, _W), jnp.float32),
            pltpu.VMEM((chunk,), jnp.float32),
            pltpu.VMEM((chunk * K,), jnp.float32),
            pltpu.SemaphoreType.DMA,
        ],
    )
    def phase_b(uids_h, rows_h, uemb_h, pos_o, neg_o,
                idx_u, rows_u, rows_p, rows_n, pos_v, neg_v, sem):
        wid = lax.axis_index("s") * NC + lax.axis_index("c")
        pltpu.sync_copy(uids_h.at[wid], idx_u)

        for s in range(n_sub):
            cps = [
                pltpu.async_copy(uemb_h.at[idx_u.at[s]], rows_u, sem),
                pltpu.async_copy(
                    rows_h.at[pl.ds(wid * chunk + s * _SUB, _SUB)],
                    rows_p, sem),
                pltpu.async_copy(
                    rows_h.at[pl.ds(B + (wid * chunk + s * _SUB) * K,
                                    _SUB * K)],
                    rows_n, sem),
            ]
            for c in cps:
                c.wait()

            def group(g, carry):
                bloc = g * _LANES + lax.iota(jnp.int32, _LANES)
                babs = s * _SUB + bloc
                accp = jnp.zeros((_LANES,), jnp.float32)
                accn = [jnp.zeros((_LANES,), jnp.float32) for _ in range(K)]
                nrow = [bloc * K + k for k in range(K)]
                for d in range(D):
                    col = jnp.full((_LANES,), d, jnp.int32)
                    uv = plsc.load_gather(rows_u, [bloc, col])
                    pv = plsc.load_gather(rows_p, [bloc, col])
                    accp = accp + uv * pv
                    for k in range(K):
                        nv = plsc.load_gather(rows_n, [nrow[k], col])
                        accn[k] = accn[k] + uv * nv
                plsc.store_scatter(pos_v, [babs], accp)
                for k in range(K):
                    plsc.store_scatter(neg_v, [babs * K + k], accn[k])
                return carry

            lax.fori_loop(0, n_grp, group, 0)

        pltpu.sync_copy(pos_v, pos_o.at[pl.ds(wid * chunk, chunk)])
        pltpu.sync_copy(neg_v, neg_o.at[pl.ds(wid * chunk * K, chunk * K)])

    return phase_b(uids, rows_book, user_pad)


def _loss_tc(pos_s, neg_s, B):
    pos2 = pos_s.reshape(-1, 128)
    neg2 = neg_s.reshape(-1, 128)

    def body(p_ref, n_ref, o_ref):
        p = p_ref[...]
        n = n_ref[...]
        lp = jnp.log(1.0 / (1.0 + jnp.exp(-p)) + 1e-10)
        ln = jnp.log(1.0 / (1.0 + jnp.exp(n)) + 1e-10)
        o_ref[0, 0] = -(jnp.sum(lp) + jnp.sum(ln)) * (1.0 / B)

    out = pl.pallas_call(
        body,
        out_shape=jax.ShapeDtypeStruct((1, 1), jnp.float32),
        out_specs=pl.BlockSpec(memory_space=pltpu.SMEM),
    )(pos2, neg2)
    return out[0, 0]


def kernel(user_embed, book_embed, user_ids, pos_book_ids, neg_book_ids):
    B = user_ids.shape[0]
    K = neg_book_ids.shape[1]
    V, D = book_embed.shape
    info = plsc.get_sparse_core_info()
    NW = info.num_cores * info.num_subcores
    uids = user_ids.astype(jnp.int32).reshape(NW, -1, _IDXW)
    allids = jnp.concatenate([
        pos_book_ids.astype(jnp.int32),
        neg_book_ids.astype(jnp.int32).reshape(-1),
    ]).reshape(-1, _IDXW)
    n_ids = B * (K + 1)
    user_pad = jnp.pad(user_embed, ((0, 0), (0, _W - D)))
    rows_book = _sc_gather_book(book_embed.T, allids, n_ids, V, D)
    pos_s, neg_s = _sc_scores(rows_book, user_pad, uids, B, K, D)
    return _loss_tc(pos_s, neg_s, B)


# speedup vs baseline: 1.3604x; 1.0036x over previous
import functools

import jax
import jax.numpy as jnp
from jax import lax
from jax.experimental import pallas as pl
from jax.experimental.pallas import tpu as pltpu
from jax.experimental.pallas import tpu_sc as plsc

_LANES = 16
_IDXW = 128
_W = 128
_SUB = 128
_BLK = 1024


def _sc_gather_book(book_t, allids, n_ids, V, D):
    info = plsc.get_sparse_core_info()
    NC, NS = info.num_cores, info.num_subcores
    NW = NC * NS
    n_chunks = (V + _BLK - 1) // _BLK
    V_pad = ((V + 127) // 128) * 128
    n_piece = n_ids // 8192
    HMAX = 4112
    TMAX = 656

    mesh = plsc.VectorSubcoreMesh(core_axis_name="c", subcore_axis_name="s")

    @functools.partial(
        pl.kernel,
        out_type=jax.ShapeDtypeStruct((n_ids + NW, _W), jnp.float32),
        mesh=mesh,
        compiler_params=pltpu.CompilerParams(
            needs_layout_passes=False,
        ),
        scratch_types=[
            pltpu.VMEM((64, _IDXW), jnp.int32),
            pltpu.VMEM((D, _BLK), jnp.float32),
            pltpu.VMEM((HMAX,), jnp.int32),
            pltpu.VMEM((HMAX,), jnp.int32),
            pltpu.VMEM((TMAX,), jnp.int32),
            pltpu.VMEM((TMAX,), jnp.int32),
            pltpu.VMEM((1, _IDXW), jnp.int32),
            pltpu.VMEM((_IDXW, _W), jnp.float32),
            pltpu.SemaphoreType.DMA,
        ],
    )
    def phase_a(ids_h, bt_h, out_h, idbuf, blk, hv, hs, tv, ts, sidx, ebuf,
                sem):
        wid = lax.axis_index("s") * NC + lax.axis_index("c")
        iota16 = lax.iota(jnp.int32, _LANES)
        start_c = (wid * n_chunks) // NW
        end_c = ((wid + 1) * n_chunks) // NW
        lo_w = start_c * _BLK
        hi_w = jnp.minimum(end_c * _BLK, V)

        def scan_piece(p, cnt):
            pltpu.sync_copy(ids_h.at[pl.ds(p * 64, 64)], idbuf)

            def scan_vec(i, cnt):
                r = i // 8
                c = (i % 8) * _LANES
                v = idbuf[r, pl.ds(c, _LANES)]
                m = jnp.logical_and(v >= lo_w, v < hi_w)
                plsc.store_compressed(hv.at[pl.ds(cnt, _LANES)], v, mask=m)
                slot = p * 8192 + i * _LANES + iota16
                plsc.store_compressed(hs.at[pl.ds(cnt, _LANES)], slot, mask=m)
                return cnt + plsc.all_reduce_population_count(m)[0]

            return lax.fori_loop(0, 512, scan_vec, cnt)

        cnt = lax.fori_loop(0, n_piece, scan_piece, jnp.int32(0))
        hv[pl.ds(cnt, _LANES)] = jnp.full((_LANES,), -1, jnp.int32)
        n_hvec = (cnt + _LANES - 1) // _LANES

        def do_chunk(c, carry):
            lo = (start_c + c) * _BLK
            hi = jnp.minimum(lo + _BLK, V)
            win = jnp.minimum(lo, jnp.int32(V_pad - _BLK))
            pltpu.sync_copy(bt_h.at[:, pl.ds(win, _BLK)], blk)

            def resc(i, c2):
                v = hv[pl.ds(i * _LANES, _LANES)]
                s = hs[pl.ds(i * _LANES, _LANES)]
                m = jnp.logical_and(v >= lo, v < hi)
                plsc.store_compressed(
                    tv.at[pl.ds(c2, _LANES)], v - win, mask=m)
                plsc.store_compressed(ts.at[pl.ds(c2, _LANES)], s, mask=m)
                return c2 + plsc.all_reduce_population_count(m)[0]

            cnt2 = lax.fori_loop(0, n_hvec, resc, jnp.int32(0))

            def padv(j, carry):
                tv[pl.ds(cnt2 + j * _LANES, _LANES)] = jnp.zeros(
                    (_LANES,), jnp.int32)
                ts[pl.ds(cnt2 + j * _LANES, _LANES)] = (
                    jnp.zeros((_LANES,), jnp.int32) + (n_ids + wid))
                return carry

            lax.fori_loop(0, 8, padv, 0)

            def batch(b, carry):
                def sub(j, carry):
                    off = b * _IDXW + j * _LANES
                    vloc = tv[pl.ds(off, _LANES)]
                    sidx[0, pl.ds(j * _LANES, _LANES)] = ts[
                        pl.ds(off, _LANES)]
                    lane = j * _LANES + iota16
                    for d in range(D):
                        dv = jnp.full((_LANES,), d, jnp.int32)
                        val = plsc.load_gather(blk, [dv, vloc])
                        plsc.store_scatter(ebuf, [lane, dv], val)
                    return carry

                lax.fori_loop(0, _IDXW // _LANES, sub, 0)
                pltpu.async_copy(ebuf, out_h.at[sidx.at[0]], sem).wait()
                return carry

            lax.fori_loop(0, (cnt2 + _IDXW - 1) // _IDXW, batch, 0)
            return carry

        lax.fori_loop(0, end_c - start_c, do_chunk, 0)

    return phase_a(allids, book_t)


def _sc_scores(rows_book, user_pad, uids, B, K, D):
    info = plsc.get_sparse_core_info()
    NC, NS = info.num_cores, info.num_subcores
    NW = NC * NS
    chunk = B // NW
    n_sub = chunk // _SUB
    n_grp = _SUB // _LANES
    n_iu = chunk // _IDXW

    mesh = plsc.VectorSubcoreMesh(core_axis_name="c", subcore_axis_name="s")

    @functools.partial(
        pl.kernel,
        out_type=[
            jax.ShapeDtypeStruct((B,), jnp.float32),
            jax.ShapeDtypeStruct((B * K,), jnp.float32),
        ],
        mesh=mesh,
        compiler_params=pltpu.CompilerParams(
            needs_layout_passes=False,
        ),
        scratch_types=[
            pltpu.VMEM((n_iu, _IDXW), jnp.int32),
            pltpu.VMEM((_SUB, _W), jnp.float32),
            pltpu.VMEM((_SUB, _W), jnp.float32),
            pltpu.VMEM((_SUB * K, _W), jnp.float32),
            pltpu.VMEM((chunk,), jnp.float32),
            pltpu.VMEM((chunk * K,), jnp.float32),
            pltpu.SemaphoreType.DMA,
        ],
    )
    def phase_b(uids_h, rows_h, uemb_h, pos_o, neg_o,
                idx_u, rows_u, rows_p, rows_n, pos_v, neg_v, sem):
        wid = lax.axis_index("s") * NC + lax.axis_index("c")
        pltpu.sync_copy(uids_h.at[wid], idx_u)

        for s in range(n_sub):
            cps = [
                pltpu.async_copy(uemb_h.at[idx_u.at[s]], rows_u, sem),
                pltpu.async_copy(
                    rows_h.at[pl.ds(wid * chunk + s * _SUB, _SUB)],
                    rows_p, sem),
                pltpu.async_copy(
                    rows_h.at[pl.ds(B + (wid * chunk + s * _SUB) * K,
                                    _SUB * K)],
                    rows_n, sem),
            ]
            for c in cps:
                c.wait()

            def group(g, carry):
                bloc = g * _LANES + lax.iota(jnp.int32, _LANES)
                babs = s * _SUB + bloc
                accp = jnp.zeros((_LANES,), jnp.float32)
                accn = [jnp.zeros((_LANES,), jnp.float32) for _ in range(K)]
                nrow = [bloc * K + k for k in range(K)]
                for d in range(D):
                    col = jnp.full((_LANES,), d, jnp.int32)
                    uv = plsc.load_gather(rows_u, [bloc, col])
                    pv = plsc.load_gather(rows_p, [bloc, col])
                    accp = accp + uv * pv
                    for k in range(K):
                        nv = plsc.load_gather(rows_n, [nrow[k], col])
                        accn[k] = accn[k] + uv * nv
                plsc.store_scatter(pos_v, [babs], accp)
                for k in range(K):
                    plsc.store_scatter(neg_v, [babs * K + k], accn[k])
                return carry

            lax.fori_loop(0, n_grp, group, 0)

        pltpu.sync_copy(pos_v, pos_o.at[pl.ds(wid * chunk, chunk)])
        pltpu.sync_copy(neg_v, neg_o.at[pl.ds(wid * chunk * K, chunk * K)])

    return phase_b(uids, rows_book, user_pad)


def _loss_tc(pos_s, neg_s, B):
    pos2 = pos_s.reshape(-1, 128)
    neg2 = neg_s.reshape(-1, 128)

    def body(p_ref, n_ref, o_ref):
        p = p_ref[...]
        n = n_ref[...]
        lp = jnp.log(1.0 / (1.0 + jnp.exp(-p)) + 1e-10)
        ln = jnp.log(1.0 / (1.0 + jnp.exp(n)) + 1e-10)
        o_ref[0, 0] = -(jnp.sum(lp) + jnp.sum(ln)) * (1.0 / B)

    out = pl.pallas_call(
        body,
        out_shape=jax.ShapeDtypeStruct((1, 1), jnp.float32),
        out_specs=pl.BlockSpec(memory_space=pltpu.SMEM),
    )(pos2, neg2)
    return out[0, 0]


def kernel(user_embed, book_embed, user_ids, pos_book_ids, neg_book_ids):
    B = user_ids.shape[0]
    K = neg_book_ids.shape[1]
    V, D = book_embed.shape
    info = plsc.get_sparse_core_info()
    NW = info.num_cores * info.num_subcores
    uids = user_ids.astype(jnp.int32).reshape(NW, -1, _IDXW)
    allids = jnp.concatenate([
        pos_book_ids.astype(jnp.int32),
        neg_book_ids.astype(jnp.int32).reshape(-1),
    ]).reshape(-1, _IDXW)
    n_ids = B * (K + 1)
    user_pad = jnp.pad(user_embed, ((0, 0), (0, _W - D)))
    rows_book = _sc_gather_book(book_embed.T, allids, n_ids, V, D)
    pos_s, neg_s = _sc_scores(rows_book, user_pad, uids, B, K, D)
    return _loss_tc(pos_s, neg_s, B)


# speedup vs baseline: 1.5164x; 1.1147x over previous
import functools

import jax
import jax.numpy as jnp
from jax import lax
from jax.experimental import pallas as pl
from jax.experimental.pallas import tpu as pltpu
from jax.experimental.pallas import tpu_sc as plsc

_LANES = 16
_IDXW = 128
_W = 128
_SUB = 128
_BLK = 2048
_NDUMP = 512


def _sc_gather_book(book_t, allids, n_ids, V, D):
    info = plsc.get_sparse_core_info()
    NC, NS = info.num_cores, info.num_subcores
    NW = NC * NS
    n_chunks = (V + _BLK - 1) // _BLK
    V_pad = ((V + 127) // 128) * 128
    n_piece = n_ids // 8192
    HMAX = 4112
    TMAX = 656

    mesh = plsc.VectorSubcoreMesh(core_axis_name="c", subcore_axis_name="s")

    @functools.partial(
        pl.kernel,
        out_type=jax.ShapeDtypeStruct((n_ids + _NDUMP, _W), jnp.float32),
        mesh=mesh,
        compiler_params=pltpu.CompilerParams(
            needs_layout_passes=False,
        ),
        scratch_types=[
            pltpu.VMEM((64, _IDXW), jnp.int32),
            pltpu.VMEM((D, _BLK), jnp.float32),
            pltpu.VMEM((HMAX,), jnp.int32),
            pltpu.VMEM((HMAX,), jnp.int32),
            pltpu.VMEM((TMAX,), jnp.int32),
            pltpu.VMEM((TMAX,), jnp.int32),
            pltpu.VMEM((1, _IDXW), jnp.int32),
            pltpu.VMEM((_IDXW, _W), jnp.float32),
            pltpu.SemaphoreType.DMA,
        ],
    )
    def phase_a(ids_h, bt_h, out_h, idbuf, blk, hv, hs, tv, ts, sidx, ebuf,
                sem):
        wid = lax.axis_index("s") * NC + lax.axis_index("c")
        iota16 = lax.iota(jnp.int32, _LANES)
        start_c = (wid * n_chunks) // NW
        end_c = ((wid + 1) * n_chunks) // NW
        lo_w = start_c * _BLK
        hi_w = jnp.minimum(end_c * _BLK, V)

        def scan_piece(p, cnt):
            pltpu.sync_copy(ids_h.at[pl.ds(p * 64, 64)], idbuf)

            def scan_vec(i, cnt):
                r = i // 8
                c = (i % 8) * _LANES
                v = idbuf[r, pl.ds(c, _LANES)]
                m = jnp.logical_and(v >= lo_w, v < hi_w)
                plsc.store_compressed(hv.at[pl.ds(cnt, _LANES)], v, mask=m)
                slot = p * 8192 + i * _LANES + iota16
                plsc.store_compressed(hs.at[pl.ds(cnt, _LANES)], slot, mask=m)
                return cnt + plsc.all_reduce_population_count(m)[0]

            return lax.fori_loop(0, 512, scan_vec, cnt)

        cnt = lax.fori_loop(0, n_piece, scan_piece, jnp.int32(0))
        hv[pl.ds(cnt, _LANES)] = jnp.full((_LANES,), -1, jnp.int32)
        n_hvec = (cnt + _LANES - 1) // _LANES

        def do_chunk(c, carry):
            lo = (start_c + c) * _BLK
            hi = jnp.minimum(lo + _BLK, V)
            win = jnp.minimum(lo, jnp.int32(V_pad - _BLK))
            pltpu.sync_copy(bt_h.at[:, pl.ds(win, _BLK)], blk)

            def resc(i, c2):
                v = hv[pl.ds(i * _LANES, _LANES)]
                s = hs[pl.ds(i * _LANES, _LANES)]
                m = jnp.logical_and(v >= lo, v < hi)
                plsc.store_compressed(
                    tv.at[pl.ds(c2, _LANES)], v - win, mask=m)
                plsc.store_compressed(ts.at[pl.ds(c2, _LANES)], s, mask=m)
                return c2 + plsc.all_reduce_population_count(m)[0]

            cnt2 = lax.fori_loop(0, n_hvec, resc, jnp.int32(0))

            def padv(j, carry):
                tv[pl.ds(cnt2 + j * _LANES, _LANES)] = jnp.zeros(
                    (_LANES,), jnp.int32)
                ts[pl.ds(cnt2 + j * _LANES, _LANES)] = (
                    iota16 + (n_ids + wid * _LANES))
                return carry

            lax.fori_loop(0, 8, padv, 0)

            def batch(b, carry):
                def sub(j, carry):
                    off = b * _IDXW + j * _LANES
                    vloc = tv[pl.ds(off, _LANES)]
                    sidx[0, pl.ds(j * _LANES, _LANES)] = ts[
                        pl.ds(off, _LANES)]
                    lane = j * _LANES + iota16
                    for d in range(D):
                        dv = jnp.full((_LANES,), d, jnp.int32)
                        val = plsc.load_gather(blk, [dv, vloc])
                        plsc.store_scatter(ebuf, [lane, dv], val)
                    return carry

                lax.fori_loop(0, _IDXW // _LANES, sub, 0)
                pltpu.async_copy(ebuf, out_h.at[sidx.at[0]], sem).wait()
                return carry

            lax.fori_loop(0, (cnt2 + _IDXW - 1) // _IDXW, batch, 0)
            return carry

        lax.fori_loop(0, end_c - start_c, do_chunk, 0)

    return phase_a(allids, book_t)


def _sc_scores(rows_book, user_pad, uids, B, K, D):
    info = plsc.get_sparse_core_info()
    NC, NS = info.num_cores, info.num_subcores
    NW = NC * NS
    chunk = B // NW
    n_sub = chunk // _SUB
    n_grp = _SUB // _LANES
    n_iu = chunk // _IDXW

    mesh = plsc.VectorSubcoreMesh(core_axis_name="c", subcore_axis_name="s")

    @functools.partial(
        pl.kernel,
        out_type=[
            jax.ShapeDtypeStruct((B,), jnp.float32),
            jax.ShapeDtypeStruct((B * K,), jnp.float32),
        ],
        mesh=mesh,
        compiler_params=pltpu.CompilerParams(
            needs_layout_passes=False,
        ),
        scratch_types=[
            pltpu.VMEM((n_iu, _IDXW), jnp.int32),
            pltpu.VMEM((_SUB, _W), jnp.float32),
            pltpu.VMEM((_SUB, _W), jnp.float32),
            pltpu.VMEM((_SUB * K, _W), jnp.float32),
            pltpu.VMEM((chunk,), jnp.float32),
            pltpu.VMEM((chunk * K,), jnp.float32),
            pltpu.SemaphoreType.DMA,
        ],
    )
    def phase_b(uids_h, rows_h, uemb_h, pos_o, neg_o,
                idx_u, rows_u, rows_p, rows_n, pos_v, neg_v, sem):
        wid = lax.axis_index("s") * NC + lax.axis_index("c")
        pltpu.sync_copy(uids_h.at[wid], idx_u)

        for s in range(n_sub):
            cps = [
                pltpu.async_copy(uemb_h.at[idx_u.at[s]], rows_u, sem),
                pltpu.async_copy(
                    rows_h.at[pl.ds(wid * chunk + s * _SUB, _SUB)],
                    rows_p, sem),
                pltpu.async_copy(
                    rows_h.at[pl.ds(B + (wid * chunk + s * _SUB) * K,
                                    _SUB * K)],
                    rows_n, sem),
            ]
            for c in cps:
                c.wait()

            def group(g, carry):
                bloc = g * _LANES + lax.iota(jnp.int32, _LANES)
                babs = s * _SUB + bloc
                accp = jnp.zeros((_LANES,), jnp.float32)
                accn = [jnp.zeros((_LANES,), jnp.float32) for _ in range(K)]
                nrow = [bloc * K + k for k in range(K)]
                for d in range(D):
                    col = jnp.full((_LANES,), d, jnp.int32)
                    uv = plsc.load_gather(rows_u, [bloc, col])
                    pv = plsc.load_gather(rows_p, [bloc, col])
                    accp = accp + uv * pv
                    for k in range(K):
                        nv = plsc.load_gather(rows_n, [nrow[k], col])
                        accn[k] = accn[k] + uv * nv
                plsc.store_scatter(pos_v, [babs], accp)
                for k in range(K):
                    plsc.store_scatter(neg_v, [babs * K + k], accn[k])
                return carry

            lax.fori_loop(0, n_grp, group, 0)

        pltpu.sync_copy(pos_v, pos_o.at[pl.ds(wid * chunk, chunk)])
        pltpu.sync_copy(neg_v, neg_o.at[pl.ds(wid * chunk * K, chunk * K)])

    return phase_b(uids, rows_book, user_pad)


def _loss_tc(pos_s, neg_s, B):
    pos2 = pos_s.reshape(-1, 128)
    neg2 = neg_s.reshape(-1, 128)

    def body(p_ref, n_ref, o_ref):
        p = p_ref[...]
        n = n_ref[...]
        lp = jnp.log(1.0 / (1.0 + jnp.exp(-p)) + 1e-10)
        ln = jnp.log(1.0 / (1.0 + jnp.exp(n)) + 1e-10)
        o_ref[0, 0] = -(jnp.sum(lp) + jnp.sum(ln)) * (1.0 / B)

    out = pl.pallas_call(
        body,
        out_shape=jax.ShapeDtypeStruct((1, 1), jnp.float32),
        out_specs=pl.BlockSpec(memory_space=pltpu.SMEM),
    )(pos2, neg2)
    return out[0, 0]


def kernel(user_embed, book_embed, user_ids, pos_book_ids, neg_book_ids):
    B = user_ids.shape[0]
    K = neg_book_ids.shape[1]
    V, D = book_embed.shape
    info = plsc.get_sparse_core_info()
    NW = info.num_cores * info.num_subcores
    uids = user_ids.astype(jnp.int32).reshape(NW, -1, _IDXW)
    allids = jnp.concatenate([
        pos_book_ids.astype(jnp.int32),
        neg_book_ids.astype(jnp.int32).reshape(-1),
    ]).reshape(-1, _IDXW)
    n_ids = B * (K + 1)
    user_pad = jnp.pad(user_embed, ((0, 0), (0, _W - D)))
    rows_book = _sc_gather_book(book_embed.T, allids, n_ids, V, D)
    pos_s, neg_s = _sc_scores(rows_book, user_pad, uids, B, K, D)
    return _loss_tc(pos_s, neg_s, B)


# speedup vs baseline: 1.5591x; 1.0282x over previous
import functools

import jax
import jax.numpy as jnp
from jax import lax
from jax.experimental import pallas as pl
from jax.experimental.pallas import tpu as pltpu
from jax.experimental.pallas import tpu_sc as plsc

_LANES = 16
_IDXW = 128
_W = 128
_SUB = 128
_BLK = 2048
_NDUMP = 512


def _sc_gather_book(book_t, allids, n_ids, V, D):
    info = plsc.get_sparse_core_info()
    NC, NS = info.num_cores, info.num_subcores
    NW = NC * NS
    n_chunks = (V + _BLK - 1) // _BLK
    V_pad = ((V + 127) // 128) * 128
    n_piece = n_ids // 8192
    HMAX = 4112
    TMAX = 656

    mesh = plsc.VectorSubcoreMesh(core_axis_name="c", subcore_axis_name="s")

    @functools.partial(
        pl.kernel,
        out_type=jax.ShapeDtypeStruct((n_ids + _NDUMP, _W), jnp.float32),
        mesh=mesh,
        compiler_params=pltpu.CompilerParams(
            needs_layout_passes=False,
        ),
        scratch_types=[
            pltpu.VMEM((64, _IDXW), jnp.int32),
            pltpu.VMEM((D, _BLK), jnp.float32),
            pltpu.VMEM((HMAX,), jnp.int32),
            pltpu.VMEM((HMAX,), jnp.int32),
            pltpu.VMEM((TMAX,), jnp.int32),
            pltpu.VMEM((TMAX,), jnp.int32),
            pltpu.VMEM((1, _IDXW), jnp.int32),
            pltpu.VMEM((_IDXW, _W), jnp.float32),
            pltpu.SemaphoreType.DMA,
        ],
    )
    def phase_a(ids_h, bt_h, out_h, idbuf, blk, hv, hs, tv, ts, sidx, ebuf,
                sem):
        wid = lax.axis_index("s") * NC + lax.axis_index("c")
        iota16 = lax.iota(jnp.int32, _LANES)
        start_c = (wid * n_chunks) // NW
        end_c = ((wid + 1) * n_chunks) // NW
        lo_w = start_c * _BLK
        hi_w = jnp.minimum(end_c * _BLK, V)

        def scan_piece(p, cnt):
            pltpu.sync_copy(ids_h.at[pl.ds(p * 64, 64)], idbuf)

            def scan_vec(i, cnt):
                r = i // 8
                c = (i % 8) * _LANES
                v = idbuf[r, pl.ds(c, _LANES)]
                m = jnp.logical_and(v >= lo_w, v < hi_w)
                plsc.store_compressed(hv.at[pl.ds(cnt, _LANES)], v, mask=m)
                slot = p * 8192 + i * _LANES + iota16
                plsc.store_compressed(hs.at[pl.ds(cnt, _LANES)], slot, mask=m)
                return cnt + plsc.all_reduce_population_count(m)[0]

            return lax.fori_loop(0, 512, scan_vec, cnt)

        cnt = lax.fori_loop(0, n_piece, scan_piece, jnp.int32(0))
        hv[pl.ds(cnt, _LANES)] = jnp.full((_LANES,), -1, jnp.int32)
        n_hvec = (cnt + _LANES - 1) // _LANES

        def do_chunk(c, carry):
            lo = (start_c + c) * _BLK
            hi = jnp.minimum(lo + _BLK, V)
            win = jnp.minimum(lo, jnp.int32(V_pad - _BLK))
            pltpu.sync_copy(bt_h.at[:, pl.ds(win, _BLK)], blk)

            def resc(i, c2):
                v = hv[pl.ds(i * _LANES, _LANES)]
                s = hs[pl.ds(i * _LANES, _LANES)]
                m = jnp.logical_and(v >= lo, v < hi)
                plsc.store_compressed(
                    tv.at[pl.ds(c2, _LANES)], v - win, mask=m)
                plsc.store_compressed(ts.at[pl.ds(c2, _LANES)], s, mask=m)
                return c2 + plsc.all_reduce_population_count(m)[0]

            cnt2 = lax.fori_loop(0, n_hvec, resc, jnp.int32(0))

            def padv(j, carry):
                tv[pl.ds(cnt2 + j * _LANES, _LANES)] = jnp.zeros(
                    (_LANES,), jnp.int32)
                ts[pl.ds(cnt2 + j * _LANES, _LANES)] = (
                    iota16 + (n_ids + wid * _LANES))
                return carry

            lax.fori_loop(0, 8, padv, 0)

            def batch(b, carry):
                def sub(j, carry):
                    off = b * _IDXW + j * _LANES
                    vloc = tv[pl.ds(off, _LANES)]
                    sidx[0, pl.ds(j * _LANES, _LANES)] = ts[
                        pl.ds(off, _LANES)]
                    lane = j * _LANES + iota16
                    for d in range(D):
                        dv = jnp.full((_LANES,), d, jnp.int32)
                        val = plsc.load_gather(blk, [dv, vloc])
                        plsc.store_scatter(ebuf, [lane, dv], val)
                    return carry

                lax.fori_loop(0, _IDXW // _LANES, sub, 0)
                pltpu.async_copy(ebuf, out_h.at[sidx.at[0]], sem).wait()
                return carry

            lax.fori_loop(0, (cnt2 + _IDXW - 1) // _IDXW, batch, 0)
            return carry

        lax.fori_loop(0, end_c - start_c, do_chunk, 0)

    return phase_a(allids, book_t)


def _sc_scores(rows_book, user_pad, uids, B, K, D):
    info = plsc.get_sparse_core_info()
    NC, NS = info.num_cores, info.num_subcores
    NW = NC * NS
    SUBB = 64
    chunk = B // NW
    n_sub = chunk // SUBB
    n_grp = SUBB // _LANES
    n_iu = chunk // _IDXW

    mesh = plsc.VectorSubcoreMesh(core_axis_name="c", subcore_axis_name="s")

    @functools.partial(
        pl.kernel,
        out_type=[
            jax.ShapeDtypeStruct((B,), jnp.float32),
            jax.ShapeDtypeStruct((B * K,), jnp.float32),
        ],
        mesh=mesh,
        compiler_params=pltpu.CompilerParams(
            needs_layout_passes=False,
        ),
        scratch_types=[
            pltpu.VMEM((n_iu, _IDXW), jnp.int32),
            pltpu.VMEM((SUBB, _W), jnp.float32),
            pltpu.VMEM((SUBB, _W), jnp.float32),
            pltpu.VMEM((SUBB * K, _W), jnp.float32),
            pltpu.VMEM((SUBB, _W), jnp.float32),
            pltpu.VMEM((SUBB, _W), jnp.float32),
            pltpu.VMEM((SUBB * K, _W), jnp.float32),
            pltpu.VMEM((chunk,), jnp.float32),
            pltpu.VMEM((chunk * K,), jnp.float32),
            pltpu.SemaphoreType.DMA,
            pltpu.SemaphoreType.DMA,
        ],
    )
    def phase_b(uids_h, rows_h, uemb_h, pos_o, neg_o,
                idx_u, ru0, rp0, rn0, ru1, rp1, rn1, pos_v, neg_v,
                sem0, sem1):
        wid = lax.axis_index("s") * NC + lax.axis_index("c")
        pltpu.sync_copy(uids_h.at[wid], idx_u)
        bufs = [(ru0, rp0, rn0), (ru1, rp1, rn1)]
        sems = [sem0, sem1]

        def fire(s):
            ru, rp, rn = bufs[s % 2]
            sm = sems[s % 2]
            uidx = idx_u.at[s // 2, pl.ds((s % 2) * SUBB, SUBB)]
            return [
                pltpu.async_copy(uemb_h.at[uidx], ru, sm),
                pltpu.async_copy(
                    rows_h.at[pl.ds(wid * chunk + s * SUBB, SUBB)], rp, sm),
                pltpu.async_copy(
                    rows_h.at[pl.ds(B + (wid * chunk + s * SUBB) * K,
                                    SUBB * K)], rn, sm),
            ]

        pend = {0: fire(0)}
        for s in range(n_sub):
            if s + 1 < n_sub:
                pend[s + 1] = fire(s + 1)
            for c in pend.pop(s):
                c.wait()
            rows_u, rows_p, rows_n = bufs[s % 2]

            def group(g, carry, s=s, rows_u=rows_u, rows_p=rows_p,
                      rows_n=rows_n):
                bloc = g * _LANES + lax.iota(jnp.int32, _LANES)
                babs = s * SUBB + bloc
                accp = jnp.zeros((_LANES,), jnp.float32)
                accn = [jnp.zeros((_LANES,), jnp.float32) for _ in range(K)]
                nrow = [bloc * K + k for k in range(K)]
                for d in range(D):
                    col = jnp.full((_LANES,), d, jnp.int32)
                    uv = plsc.load_gather(rows_u, [bloc, col])
                    pv = plsc.load_gather(rows_p, [bloc, col])
                    accp = accp + uv * pv
                    for k in range(K):
                        nv = plsc.load_gather(rows_n, [nrow[k], col])
                        accn[k] = accn[k] + uv * nv
                plsc.store_scatter(pos_v, [babs], accp)
                for k in range(K):
                    plsc.store_scatter(neg_v, [babs * K + k], accn[k])
                return carry

            lax.fori_loop(0, n_grp, group, 0)

        pltpu.sync_copy(pos_v, pos_o.at[pl.ds(wid * chunk, chunk)])
        pltpu.sync_copy(neg_v, neg_o.at[pl.ds(wid * chunk * K, chunk * K)])

    return phase_b(uids, rows_book, user_pad)


def _loss_tc(pos_s, neg_s, B):
    pos2 = pos_s.reshape(-1, 128)
    neg2 = neg_s.reshape(-1, 128)

    def body(p_ref, n_ref, o_ref):
        p = p_ref[...]
        n = n_ref[...]
        lp = jnp.log(1.0 / (1.0 + jnp.exp(-p)) + 1e-10)
        ln = jnp.log(1.0 / (1.0 + jnp.exp(n)) + 1e-10)
        o_ref[0, 0] = -(jnp.sum(lp) + jnp.sum(ln)) * (1.0 / B)

    out = pl.pallas_call(
        body,
        out_shape=jax.ShapeDtypeStruct((1, 1), jnp.float32),
        out_specs=pl.BlockSpec(memory_space=pltpu.SMEM),
    )(pos2, neg2)
    return out[0, 0]


def kernel(user_embed, book_embed, user_ids, pos_book_ids, neg_book_ids):
    B = user_ids.shape[0]
    K = neg_book_ids.shape[1]
    V, D = book_embed.shape
    info = plsc.get_sparse_core_info()
    NW = info.num_cores * info.num_subcores
    uids = user_ids.astype(jnp.int32).reshape(NW, -1, _IDXW)
    allids = jnp.concatenate([
        pos_book_ids.astype(jnp.int32),
        neg_book_ids.astype(jnp.int32).reshape(-1),
    ]).reshape(-1, _IDXW)
    n_ids = B * (K + 1)
    user_pad = jnp.pad(user_embed, ((0, 0), (0, _W - D)))
    rows_book = _sc_gather_book(book_embed.T, allids, n_ids, V, D)
    pos_s, neg_s = _sc_scores(rows_book, user_pad, uids, B, K, D)
    return _loss_tc(pos_s, neg_s, B)


# speedup vs baseline: 1.5778x; 1.0120x over previous
import functools

import jax
import jax.numpy as jnp
from jax import lax
from jax.experimental import pallas as pl
from jax.experimental.pallas import tpu as pltpu
from jax.experimental.pallas import tpu_sc as plsc

_LANES = 16
_IDXW = 128
_W = 128
_SUB = 128
_BLK = 1280
_NDUMP = 512


def _sc_gather_book(book_t, allids, n_ids, V, D):
    info = plsc.get_sparse_core_info()
    NC, NS = info.num_cores, info.num_subcores
    NW = NC * NS
    n_chunks = (V + _BLK - 1) // _BLK
    V_pad = ((V + 127) // 128) * 128
    n_piece = n_ids // 8192
    HMAX = 4112
    TMAX = 656

    mesh = plsc.VectorSubcoreMesh(core_axis_name="c", subcore_axis_name="s")

    @functools.partial(
        pl.kernel,
        out_type=jax.ShapeDtypeStruct((n_ids + _NDUMP, _W), jnp.float32),
        mesh=mesh,
        compiler_params=pltpu.CompilerParams(
            needs_layout_passes=False,
        ),
        scratch_types=[
            pltpu.VMEM((64, _IDXW), jnp.int32),
            pltpu.VMEM((D, _BLK), jnp.float32),
            pltpu.VMEM((D, _BLK), jnp.float32),
            pltpu.VMEM((HMAX,), jnp.int32),
            pltpu.VMEM((HMAX,), jnp.int32),
            pltpu.VMEM((TMAX,), jnp.int32),
            pltpu.VMEM((TMAX,), jnp.int32),
            pltpu.VMEM((1, _IDXW), jnp.int32),
            pltpu.VMEM((_IDXW, _W), jnp.float32),
            pltpu.SemaphoreType.DMA,
            pltpu.SemaphoreType.DMA,
            pltpu.SemaphoreType.DMA,
        ],
    )
    def phase_a(ids_h, bt_h, out_h, idbuf, blk0, blk1, hv, hs, tv, ts, sidx,
                ebuf, semb0, semb1, sem_sc):
        wid = lax.axis_index("s") * NC + lax.axis_index("c")
        iota16 = lax.iota(jnp.int32, _LANES)
        start_c = (wid * n_chunks) // NW
        end_c = ((wid + 1) * n_chunks) // NW
        lo_w = start_c * _BLK
        hi_w = jnp.minimum(end_c * _BLK, V)

        def scan_piece(p, cnt):
            pltpu.sync_copy(ids_h.at[pl.ds(p * 64, 64)], idbuf)

            def scan_vec(i, cnt):
                r = i // 8
                c = (i % 8) * _LANES
                v = idbuf[r, pl.ds(c, _LANES)]
                m = jnp.logical_and(v >= lo_w, v < hi_w)
                plsc.store_compressed(hv.at[pl.ds(cnt, _LANES)], v, mask=m)
                slot = p * 8192 + i * _LANES + iota16
                plsc.store_compressed(hs.at[pl.ds(cnt, _LANES)], slot, mask=m)
                return cnt + plsc.all_reduce_population_count(m)[0]

            return lax.fori_loop(0, 512, scan_vec, cnt)

        cnt = lax.fori_loop(0, n_piece, scan_piece, jnp.int32(0))
        hv[pl.ds(cnt, _LANES)] = jnp.full((_LANES,), -1, jnp.int32)
        n_hvec = (cnt + _LANES - 1) // _LANES

        nch = end_c - start_c
        blks = [blk0, blk1]
        semsb = [semb0, semb1]

        def win_of(c):
            return jnp.minimum(
                (start_c + c) * _BLK, jnp.int32(V_pad - _BLK))

        def fire(c, p):
            return pltpu.async_copy(
                bt_h.at[:, pl.ds(win_of(c), _BLK)], blks[p], semsb[p])

        def process(c, blk, sm):
            lo = (start_c + c) * _BLK
            hi = jnp.minimum(lo + _BLK, V)
            win = win_of(c)
            pltpu.make_async_copy(
                bt_h.at[:, pl.ds(win, _BLK)], blk, sm).wait()

            def resc(i, c2):
                v = hv[pl.ds(i * _LANES, _LANES)]
                s = hs[pl.ds(i * _LANES, _LANES)]
                m = jnp.logical_and(v >= lo, v < hi)
                plsc.store_compressed(
                    tv.at[pl.ds(c2, _LANES)], v - win, mask=m)
                plsc.store_compressed(ts.at[pl.ds(c2, _LANES)], s, mask=m)
                return c2 + plsc.all_reduce_population_count(m)[0]

            cnt2 = lax.fori_loop(0, n_hvec, resc, jnp.int32(0))

            def padv(j, carry):
                tv[pl.ds(cnt2 + j * _LANES, _LANES)] = jnp.zeros(
                    (_LANES,), jnp.int32)
                ts[pl.ds(cnt2 + j * _LANES, _LANES)] = (
                    iota16 + (n_ids + wid * _LANES))
                return carry

            lax.fori_loop(0, 8, padv, 0)

            def batch(b, carry):
                def sub(j, carry):
                    off = b * _IDXW + j * _LANES
                    vloc = tv[pl.ds(off, _LANES)]
                    sidx[0, pl.ds(j * _LANES, _LANES)] = ts[
                        pl.ds(off, _LANES)]
                    lane = j * _LANES + iota16
                    for d in range(D):
                        dv = jnp.full((_LANES,), d, jnp.int32)
                        val = plsc.load_gather(blk, [dv, vloc])
                        plsc.store_scatter(ebuf, [lane, dv], val)
                    return carry

                lax.fori_loop(0, _IDXW // _LANES, sub, 0)
                pltpu.async_copy(ebuf, out_h.at[sidx.at[0]], sem_sc).wait()
                return carry

            lax.fori_loop(0, (cnt2 + _IDXW - 1) // _IDXW, batch, 0)

        fire(0, 0)

        def do_chunk(c, carry):
            even = lax.rem(c, 2) == 0

            @pl.when(jnp.logical_and(c + 1 < nch, even))
            def _():
                fire(c + 1, 1)

            @pl.when(jnp.logical_and(c + 1 < nch, jnp.logical_not(even)))
            def _():
                fire(c + 1, 0)

            @pl.when(even)
            def _():
                process(c, blk0, semb0)

            @pl.when(jnp.logical_not(even))
            def _():
                process(c, blk1, semb1)

            return carry

        lax.fori_loop(0, nch, do_chunk, 0)

    return phase_a(allids, book_t)


def _sc_scores(rows_book, user_pad, uids, B, K, D):
    info = plsc.get_sparse_core_info()
    NC, NS = info.num_cores, info.num_subcores
    NW = NC * NS
    SUBB = 64
    chunk = B // NW
    n_sub = chunk // SUBB
    n_grp = SUBB // _LANES
    n_iu = chunk // _IDXW

    mesh = plsc.VectorSubcoreMesh(core_axis_name="c", subcore_axis_name="s")

    @functools.partial(
        pl.kernel,
        out_type=[
            jax.ShapeDtypeStruct((B,), jnp.float32),
            jax.ShapeDtypeStruct((B * K,), jnp.float32),
        ],
        mesh=mesh,
        compiler_params=pltpu.CompilerParams(
            needs_layout_passes=False,
        ),
        scratch_types=[
            pltpu.VMEM((n_iu, _IDXW), jnp.int32),
            pltpu.VMEM((SUBB, _W), jnp.float32),
            pltpu.VMEM((SUBB, _W), jnp.float32),
            pltpu.VMEM((SUBB * K, _W), jnp.float32),
            pltpu.VMEM((SUBB, _W), jnp.float32),
            pltpu.VMEM((SUBB, _W), jnp.float32),
            pltpu.VMEM((SUBB * K, _W), jnp.float32),
            pltpu.VMEM((chunk,), jnp.float32),
            pltpu.VMEM((chunk * K,), jnp.float32),
            pltpu.SemaphoreType.DMA,
            pltpu.SemaphoreType.DMA,
        ],
    )
    def phase_b(uids_h, rows_h, uemb_h, pos_o, neg_o,
                idx_u, ru0, rp0, rn0, ru1, rp1, rn1, pos_v, neg_v,
                sem0, sem1):
        wid = lax.axis_index("s") * NC + lax.axis_index("c")
        pltpu.sync_copy(uids_h.at[wid], idx_u)
        bufs = [(ru0, rp0, rn0), (ru1, rp1, rn1)]
        sems = [sem0, sem1]

        def fire(s):
            ru, rp, rn = bufs[s % 2]
            sm = sems[s % 2]
            uidx = idx_u.at[s // 2, pl.ds((s % 2) * SUBB, SUBB)]
            return [
                pltpu.async_copy(uemb_h.at[uidx], ru, sm),
                pltpu.async_copy(
                    rows_h.at[pl.ds(wid * chunk + s * SUBB, SUBB)], rp, sm),
                pltpu.async_copy(
                    rows_h.at[pl.ds(B + (wid * chunk + s * SUBB) * K,
                                    SUBB * K)], rn, sm),
            ]

        pend = {0: fire(0)}
        for s in range(n_sub):
            if s + 1 < n_sub:
                pend[s + 1] = fire(s + 1)
            for c in pend.pop(s):
                c.wait()
            rows_u, rows_p, rows_n = bufs[s % 2]

            def group(g, carry, s=s, rows_u=rows_u, rows_p=rows_p,
                      rows_n=rows_n):
                bloc = g * _LANES + lax.iota(jnp.int32, _LANES)
                babs = s * SUBB + bloc
                accp = jnp.zeros((_LANES,), jnp.float32)
                accn = [jnp.zeros((_LANES,), jnp.float32) for _ in range(K)]
                nrow = [bloc * K + k for k in range(K)]
                for d in range(D):
                    col = jnp.full((_LANES,), d, jnp.int32)
                    uv = plsc.load_gather(rows_u, [bloc, col])
                    pv = plsc.load_gather(rows_p, [bloc, col])
                    accp = accp + uv * pv
                    for k in range(K):
                        nv = plsc.load_gather(rows_n, [nrow[k], col])
                        accn[k] = accn[k] + uv * nv
                plsc.store_scatter(pos_v, [babs], accp)
                for k in range(K):
                    plsc.store_scatter(neg_v, [babs * K + k], accn[k])
                return carry

            lax.fori_loop(0, n_grp, group, 0)

        pltpu.sync_copy(pos_v, pos_o.at[pl.ds(wid * chunk, chunk)])
        pltpu.sync_copy(neg_v, neg_o.at[pl.ds(wid * chunk * K, chunk * K)])

    return phase_b(uids, rows_book, user_pad)


def _loss_tc(pos_s, neg_s, B):
    pos2 = pos_s.reshape(-1, 128)
    neg2 = neg_s.reshape(-1, 128)

    def body(p_ref, n_ref, o_ref):
        p = p_ref[...]
        n = n_ref[...]
        lp = jnp.log(1.0 / (1.0 + jnp.exp(-p)) + 1e-10)
        ln = jnp.log(1.0 / (1.0 + jnp.exp(n)) + 1e-10)
        o_ref[0, 0] = -(jnp.sum(lp) + jnp.sum(ln)) * (1.0 / B)

    out = pl.pallas_call(
        body,
        out_shape=jax.ShapeDtypeStruct((1, 1), jnp.float32),
        out_specs=pl.BlockSpec(memory_space=pltpu.SMEM),
    )(pos2, neg2)
    return out[0, 0]


def kernel(user_embed, book_embed, user_ids, pos_book_ids, neg_book_ids):
    B = user_ids.shape[0]
    K = neg_book_ids.shape[1]
    V, D = book_embed.shape
    info = plsc.get_sparse_core_info()
    NW = info.num_cores * info.num_subcores
    uids = user_ids.astype(jnp.int32).reshape(NW, -1, _IDXW)
    allids = jnp.concatenate([
        pos_book_ids.astype(jnp.int32),
        neg_book_ids.astype(jnp.int32).reshape(-1),
    ]).reshape(-1, _IDXW)
    n_ids = B * (K + 1)
    user_pad = jnp.pad(user_embed, ((0, 0), (0, _W - D)))
    rows_book = _sc_gather_book(book_embed.T, allids, n_ids, V, D)
    pos_s, neg_s = _sc_scores(rows_book, user_pad, uids, B, K, D)
    return _loss_tc(pos_s, neg_s, B)
